# baseline (speedup 1.0000x reference)
import functools

import jax
import jax.numpy as jnp
from jax import lax
from jax.experimental import pallas as pl
from jax.experimental.pallas import tpu as pltpu
from jax.experimental.pallas import tpu_sc as plsc

_N = 16384
_C = 1000

_INFO = plsc.get_sparse_core_info()
_NC = _INFO.num_cores
_NS = _INFO.num_subcores
_NW = _NC * _NS
_RPW = _N // _NW
_G = _RPW // 16

_mesh = plsc.VectorSubcoreMesh(core_axis_name="c", subcore_axis_name="s")


@functools.partial(
    pl.kernel,
    mesh=_mesh,
    out_type=jax.ShapeDtypeStruct((_NW, 16), jnp.float32),
    scratch_types=[
        pltpu.VMEM((16 * _C,), jnp.float32),
        pltpu.VMEM((_RPW,), jnp.int32),
        pltpu.VMEM((16,), jnp.float32),
    ],
    compiler_params=pltpu.CompilerParams(use_tc_tiling_on_sc=False, needs_layout_passes=False),
)
def _sc_recall(true_hbm, logits_hbm, out_hbm, buf, tvec, cnt_v):
    wid = lax.axis_index("s") * _NC + lax.axis_index("c")
    base = wid * _RPW
    pltpu.sync_copy(true_hbm.at[pl.ds(base, _RPW)], tvec)

    row_off = lax.broadcasted_iota(jnp.int32, (16,), 0) * _C

    def group_body(g, acc):
        pltpu.sync_copy(logits_hbm.at[pl.ds((base + g * 16) * _C, 16 * _C)], buf)
        m0 = plsc.load_gather(buf, [row_off])
        am0 = jnp.zeros((16,), jnp.int32)

        def col_body(cidx, carry):
            m, am = carry
            cvec = jnp.full((16,), cidx, jnp.int32)
            v = plsc.load_gather(buf, [row_off + cvec])
            upd = v > m
            m = jnp.where(upd, v, m)
            am = jnp.where(upd, cvec, am)
            return m, am

        _, am = lax.fori_loop(1, _C, col_body, (m0, am0))
        t16 = tvec[pl.ds(g * 16, 16)]
        return acc + jnp.where(am == t16, 1, 0)

    acc = lax.fori_loop(0, _G, group_body, jnp.zeros((16,), jnp.int32))
    cnt_v[...] = acc.astype(jnp.float32)
    pltpu.sync_copy(cnt_v, out_hbm.at[wid])


def kernel(true, logits):
    parts = _sc_recall(true.astype(jnp.int32), logits.reshape(_N * _C))
    return jnp.sum(parts) * (1.0 / _N)


# speedup vs baseline: 1.5594x; 1.5594x over previous
import functools

import jax
import jax.numpy as jnp
from jax import lax
from jax.experimental import pallas as pl
from jax.experimental.pallas import tpu as pltpu
from jax.experimental.pallas import tpu_sc as plsc

_N = 16384
_C = 1000

_INFO = plsc.get_sparse_core_info()
_NC = _INFO.num_cores
_NS = _INFO.num_subcores
_NW = _NC * _NS
_RPW = _N // _NW
_G = _RPW // 16

_mesh = plsc.VectorSubcoreMesh(core_axis_name="c", subcore_axis_name="s")


@functools.partial(
    pl.kernel,
    mesh=_mesh,
    out_type=jax.ShapeDtypeStruct((_NW, 16), jnp.float32),
    scratch_types=[
        pltpu.VMEM((16 * _C,), jnp.float32),
        pltpu.VMEM((_RPW,), jnp.int32),
        pltpu.VMEM((16,), jnp.float32),
    ],
    compiler_params=pltpu.CompilerParams(use_tc_tiling_on_sc=False, needs_layout_passes=False),
)
def _sc_recall(true_hbm, logits_hbm, out_hbm, buf, tvec, cnt_v):
    wid = lax.axis_index("s") * _NC + lax.axis_index("c")
    base = wid * _RPW
    pltpu.sync_copy(true_hbm.at[pl.ds(base, _RPW)], tvec)

    row_off = lax.broadcasted_iota(jnp.int32, (16,), 0) * _C

    ninf = jnp.full((16,), -jnp.inf, jnp.float32)
    zero = jnp.zeros((16,), jnp.int32)

    def group_body(g, acc):
        pltpu.sync_copy(logits_hbm.at[pl.ds((base + g * 16) * _C, 16 * _C)], buf)

        def col_body(i, carry):
            m0, m1, m2, m3, a0, a1, a2, a3 = carry
            for d in range(2):
                ic = 2 * i + d
                ibc = jnp.full((16,), ic, jnp.int32)
                t0 = row_off + ibc
                v0 = plsc.load_gather(buf, [t0])
                v1 = plsc.load_gather(buf, [t0 + 250])
                v2 = plsc.load_gather(buf, [t0 + 500])
                v3 = plsc.load_gather(buf, [t0 + 750])
                u0 = v0 > m0
                u1 = v1 > m1
                u2 = v2 > m2
                u3 = v3 > m3
                m0 = jnp.where(u0, v0, m0)
                m1 = jnp.where(u1, v1, m1)
                m2 = jnp.where(u2, v2, m2)
                m3 = jnp.where(u3, v3, m3)
                a0 = jnp.where(u0, ibc, a0)
                a1 = jnp.where(u1, ibc, a1)
                a2 = jnp.where(u2, ibc, a2)
                a3 = jnp.where(u3, ibc, a3)
            return m0, m1, m2, m3, a0, a1, a2, a3

        init = (ninf, ninf, ninf, ninf, zero, zero, zero, zero)
        m0, m1, m2, m3, a0, a1, a2, a3 = lax.fori_loop(0, 125, col_body, init)

        m, am = m0, a0
        for s, (ms, rs) in enumerate(((m1, a1), (m2, a2), (m3, a3)), start=1):
            u = ms > m
            m = jnp.where(u, ms, m)
            am = jnp.where(u, rs + s * 250, am)

        t16 = tvec[pl.ds(g * 16, 16)]
        return acc + jnp.where(am == t16, 1, 0)

    acc = lax.fori_loop(0, _G, group_body, jnp.zeros((16,), jnp.int32))
    cnt_v[...] = acc.astype(jnp.float32)
    pltpu.sync_copy(cnt_v, out_hbm.at[wid])


def kernel(true, logits):
    parts = _sc_recall(true.astype(jnp.int32), logits.reshape(_N * _C))
    return jnp.sum(parts) * (1.0 / _N)


# speedup vs baseline: 1.5889x; 1.0189x over previous
import functools

import jax
import jax.numpy as jnp
from jax import lax
from jax.experimental import pallas as pl
from jax.experimental.pallas import tpu as pltpu
from jax.experimental.pallas import tpu_sc as plsc

_N = 16384
_C = 1000

_INFO = plsc.get_sparse_core_info()
_NC = _INFO.num_cores
_NS = _INFO.num_subcores
_NW = _NC * _NS
_RPW = _N // _NW
_G = _RPW // 16

_mesh = plsc.VectorSubcoreMesh(core_axis_name="c", subcore_axis_name="s")


@functools.partial(
    pl.kernel,
    mesh=_mesh,
    out_type=jax.ShapeDtypeStruct((_NW, 16), jnp.float32),
    scratch_types=[
        pltpu.VMEM((16 * _C,), jnp.float32),
        pltpu.VMEM((_RPW,), jnp.int32),
        pltpu.VMEM((16,), jnp.float32),
    ],
    compiler_params=pltpu.CompilerParams(use_tc_tiling_on_sc=False, needs_layout_passes=False),
)
def _sc_recall(true_hbm, logits_hbm, out_hbm, buf, tvec, cnt_v):
    wid = lax.axis_index("s") * _NC + lax.axis_index("c")
    base = wid * _RPW
    pltpu.sync_copy(true_hbm.at[pl.ds(base, _RPW)], tvec)

    row_off = lax.broadcasted_iota(jnp.int32, (16,), 0) * _C

    ninf = jnp.full((16,), -jnp.inf, jnp.float32)
    zero = jnp.zeros((16,), jnp.int32)

    def group_body(g, acc):
        pltpu.sync_copy(logits_hbm.at[pl.ds((base + g * 16) * _C, 16 * _C)], buf)

        def col_body(i, carry):
            m0, m1, m2, m3, a0, a1, a2, a3 = carry
            for d in range(2):
                ic = 2 * i + d
                ibc = jnp.full((16,), ic, jnp.int32)
                t0 = row_off + ibc
                v0 = plsc.load_gather(buf, [t0])
                v1 = plsc.load_gather(buf, [t0 + 250])
                v2 = plsc.load_gather(buf, [t0 + 500])
                v3 = plsc.load_gather(buf, [t0 + 750])
                u0 = v0 > m0
                u1 = v1 > m1
                u2 = v2 > m2
                u3 = v3 > m3
                m0 = jnp.where(u0, v0, m0)
                m1 = jnp.where(u1, v1, m1)
                m2 = jnp.where(u2, v2, m2)
                m3 = jnp.where(u3, v3, m3)
                a0 = jnp.where(u0, ibc, a0)
                a1 = jnp.where(u1, ibc, a1)
                a2 = jnp.where(u2, ibc, a2)
                a3 = jnp.where(u3, ibc, a3)
            return m0, m1, m2, m3, a0, a1, a2, a3

        init = (ninf, ninf, ninf, ninf, zero, zero, zero, zero)
        m0, m1, m2, m3, a0, a1, a2, a3 = plsc.parallel_loop(
            0, 125, carry=init, unroll=4
        )(lambda i, carry: col_body(i, carry))

        m, am = m0, a0
        for s, (ms, rs) in enumerate(((m1, a1), (m2, a2), (m3, a3)), start=1):
            u = ms > m
            m = jnp.where(u, ms, m)
            am = jnp.where(u, rs + s * 250, am)

        t16 = tvec[pl.ds(g * 16, 16)]
        return acc + jnp.where(am == t16, 1, 0)

    acc = lax.fori_loop(0, _G, group_body, jnp.zeros((16,), jnp.int32))
    cnt_v[...] = acc.astype(jnp.float32)
    pltpu.sync_copy(cnt_v, out_hbm.at[wid])


def kernel(true, logits):
    parts = _sc_recall(true.astype(jnp.int32), logits.reshape(_N * _C))
    return jnp.sum(parts) * (1.0 / _N)
